# baseline (speedup 1.0000x reference)
import jax
import jax.numpy as jnp
from jax import lax
from jax.experimental import pallas as pl
from jax.experimental.pallas import tpu as pltpu
from jax.experimental.pallas import tpu_sc as plsc

_B, _L, _V, _D = 4096, 50, 100000, 300
_DP = 384
_DT = 304
_NC, _NS = 2, 16
_NW = _NC * _NS
_LANES = 16
_BPW = _B // _NW
_FBLOCKS = ((0, 64), (64, 64), (128, 64), (192, 64), (256, 48))


def _sc_body(ids_hbm, len_hbm, table_hbm, emb_hbm, mask_hbm,
             idx_v, len_v, mbuf, buf, tbuf, sem):
    wid = lax.axis_index("s") * _NC + lax.axis_index("c")
    b0 = wid * _BPW

    pltpu.sync_copy(len_hbm.at[pl.ds(b0, _BPW)], len_v)

    def l_body(l, carry):
        pltpu.sync_copy(ids_hbm.at[l, pl.ds(b0, _BPW)], idx_v)
        pltpu.async_copy(table_hbm.at[idx_v], buf, sem).wait()

        for j in range(_BPW // _LANES):
            lens16 = len_v[pl.ds(j * _LANES, _LANES)]
            mbuf[pl.ds(j * _LANES, _LANES)] = (l < lens16).astype(jnp.float32)
        pltpu.sync_copy(mbuf, mask_hbm.at[l, pl.ds(b0, _BPW)])

        iota = lax.iota(jnp.int32, _LANES)
        nj = _BPW // _LANES
        for fb0, fn in _FBLOCKS:
            @plsc.parallel_loop(0, (fn // _LANES) * nj, 1)
            def _transpose_tile(i, fb0=fb0):
                ct = i >> 3
                j = i & (nj - 1)
                t0 = ct * _LANES
                rows = j * _LANES + iota
                for d in range(_LANES):
                    perm = (iota + d) & (_LANES - 1)
                    v = plsc.load_gather(buf, [rows, fb0 + t0 + perm])
                    plsc.store_scatter(tbuf, [t0 + perm, rows], v)
            pltpu.sync_copy(tbuf.at[pl.ds(0, fn)],
                            emb_hbm.at[l, pl.ds(fb0, fn), pl.ds(b0, _BPW)])
        return carry

    lax.fori_loop(0, _L, l_body, 0)


@jax.jit
def _sc_call(ids_t, lens, table_pad):
    mesh = plsc.VectorSubcoreMesh(
        core_axis_name="c", subcore_axis_name="s",
        num_cores=_NC, num_subcores=_NS)
    fn = pl.kernel(
        _sc_body,
        out_type=[
            jax.ShapeDtypeStruct((_L, _DT, _B), jnp.float32),
            jax.ShapeDtypeStruct((_L, _B), jnp.float32),
        ],
        mesh=mesh,
        scratch_types=[
            pltpu.VMEM((_BPW,), jnp.int32),
            pltpu.VMEM((_BPW,), jnp.int32),
            pltpu.VMEM((_BPW,), jnp.float32),
            pltpu.VMEM((_BPW, _DP), jnp.float32),
            pltpu.VMEM((64, _BPW), jnp.float32),
            pltpu.SemaphoreType.DMA,
        ],
        compiler_params=pltpu.CompilerParams(
            needs_layout_passes=False, use_tc_tiling_on_sc=True),
    )
    return fn(ids_t, lens, table_pad)


def kernel(token_ids, lengths, fasttext_table):
    assert token_ids.shape == (_B, _L) and fasttext_table.shape == (_V, _D)
    lens = lengths.astype(jnp.int32)
    ids_t = token_ids.T.astype(jnp.int32)
    valid = jnp.arange(_L, dtype=jnp.int32)[:, None] < lens[None, :]
    ids_m = jnp.where(valid, ids_t, _V)
    table_pad = jnp.pad(fasttext_table.astype(jnp.float32),
                        ((0, 1), (0, _DP - _D)))
    emb_phys, mask_phys = _sc_call(ids_m, lens, table_pad)
    emb = jnp.transpose(emb_phys, (2, 0, 1))[:, :, :_D]
    mask = jnp.transpose(mask_phys, (1, 0))
    return emb, mask


# speedup vs baseline: 10.0833x; 10.0833x over previous
import jax
import jax.numpy as jnp
from jax import lax
from jax.experimental import pallas as pl
from jax.experimental.pallas import tpu as pltpu
from jax.experimental.pallas import tpu_sc as plsc

_B, _L, _V, _D = 4096, 50, 100000, 300
_DT = 304
_NC, _NS = 2, 16
_NW = _NC * _NS
_LANES = 16
_NG = _B // _LANES


def _sc_body(ids_hbm, len_hbm, tableT_hbm, emb_hbm, mask_hbm,
             rowbuf, ida, idb, outa, outb, lenbuf,
             sem_row, sem_a, sem_b, sem_oa, sem_ob):
    wid = lax.axis_index("s") * _NC + lax.axis_index("c")

    rowbuf[pl.ds(_V, _LANES)] = jnp.zeros((_LANES,), jnp.float32)

    pltpu.sync_copy(len_hbm, lenbuf)
    for extra in (0, _NW):
        lrow = wid + extra

        @pl.when(lrow < _L)
        def _mask_row(lrow=lrow):
            @plsc.parallel_loop(0, _NG, 1, unroll=4)
            def _m(g):
                lens16 = lenbuf[pl.ds(g * _LANES, _LANES)]
                outa[pl.ds(g * _LANES, _LANES)] = \
                    (lrow < lens16).astype(jnp.float32)

            pltpu.sync_copy(outa, mask_hbm.at[lrow])

    def _gather_row(idref, outref):
        @plsc.parallel_loop(0, _NG, 1, unroll=4)
        def _g(g):
            idx16 = idref[pl.ds(g * _LANES, _LANES)]
            outref[pl.ds(g * _LANES, _LANES)] = \
                plsc.load_gather(rowbuf, [idx16])

    nf = 9 + (wid < 12).astype(jnp.int32)

    def k_body(k, carry):
        f = wid + _NW * k
        pltpu.async_copy(ids_hbm.at[0], ida, sem_a)
        pltpu.sync_copy(tableT_hbm.at[pl.ds(f * _V, _V)],
                        rowbuf.at[pl.ds(0, _V)])

        def l2_body(l2, c2):
            l0 = l2 * 2
            pltpu.make_async_copy(ids_hbm.at[l0], ida, sem_a).wait()
            pltpu.async_copy(ids_hbm.at[l0 + 1], idb, sem_b)

            @pl.when(l2 > 0)
            def _():
                pltpu.make_async_copy(outa, emb_hbm.at[0, 0], sem_oa).wait()

            _gather_row(ida, outa)
            pltpu.async_copy(outa, emb_hbm.at[l0, f], sem_oa)
            pltpu.make_async_copy(ids_hbm.at[l0 + 1], idb, sem_b).wait()

            @pl.when(l2 < _L // 2 - 1)
            def _():
                pltpu.async_copy(ids_hbm.at[l0 + 2], ida, sem_a)

            @pl.when(l2 > 0)
            def _():
                pltpu.make_async_copy(outb, emb_hbm.at[0, 0], sem_ob).wait()

            _gather_row(idb, outb)
            pltpu.async_copy(outb, emb_hbm.at[l0 + 1, f], sem_ob)
            return c2

        lax.fori_loop(0, _L // 2, l2_body, 0)
        pltpu.make_async_copy(outa, emb_hbm.at[0, 0], sem_oa).wait()
        pltpu.make_async_copy(outb, emb_hbm.at[0, 0], sem_ob).wait()
        return carry

    lax.fori_loop(0, nf, k_body, 0)


@jax.jit
def _sc_call(ids_m, lens, tableT):
    mesh = plsc.VectorSubcoreMesh(
        core_axis_name="c", subcore_axis_name="s",
        num_cores=_NC, num_subcores=_NS)
    fn = pl.kernel(
        _sc_body,
        out_type=[
            jax.ShapeDtypeStruct((_L, _DT, _B), jnp.float32),
            jax.ShapeDtypeStruct((_L, _B), jnp.float32),
        ],
        mesh=mesh,
        scratch_types=[
            pltpu.VMEM((_V + _LANES,), jnp.float32),
            pltpu.VMEM((_B,), jnp.int32),
            pltpu.VMEM((_B,), jnp.int32),
            pltpu.VMEM((_B,), jnp.float32),
            pltpu.VMEM((_B,), jnp.float32),
            pltpu.VMEM((_B,), jnp.int32),
            pltpu.SemaphoreType.DMA,
            pltpu.SemaphoreType.DMA,
            pltpu.SemaphoreType.DMA,
            pltpu.SemaphoreType.DMA,
            pltpu.SemaphoreType.DMA,
        ],
        compiler_params=pltpu.CompilerParams(
            needs_layout_passes=False, use_tc_tiling_on_sc=True),
    )
    return fn(ids_m, lens, tableT)


def kernel(token_ids, lengths, fasttext_table):
    assert token_ids.shape == (_B, _L) and fasttext_table.shape == (_V, _D)
    lens = lengths.astype(jnp.int32)
    ids_t = token_ids.T.astype(jnp.int32)
    valid = jnp.arange(_L, dtype=jnp.int32)[:, None] < lens[None, :]
    ids_m = jnp.where(valid, ids_t, _V)
    tableT = fasttext_table.astype(jnp.float32).T.reshape(-1)
    emb_phys, mask_phys = _sc_call(ids_m, lens, tableT)
    emb = jnp.transpose(emb_phys, (2, 0, 1))[:, :, :_D]
    mask = jnp.transpose(mask_phys, (1, 0))
    return emb, mask
